# baseline (speedup 1.0000x reference)
import jax
import jax.numpy as jnp
from jax import lax
from jax.experimental import pallas as pl
from jax.experimental.pallas import tpu as pltpu
from jax.experimental.pallas import tpu_sc as plsc

VOCAB = 1000000
EMBED_DIM = 64
BATCH = 16384

NUM_CORES = 2
NUM_SUBCORES = 16
LANES = 16
NW = NUM_CORES * NUM_SUBCORES
B_PER_W = BATCH // NW
IDX_CHUNK = 128
N_CHUNKS = B_PER_W // IDX_CHUNK
D_VECS = EMBED_DIM // LANES


def _sc_scores(centers2d, contexts2d, emb):
    mesh = plsc.VectorSubcoreMesh(core_axis_name="c", subcore_axis_name="s")

    @jax.jit
    def run(centers2d, contexts2d, emb):
        @pl.kernel(
            out_type=jax.ShapeDtypeStruct((BATCH,), jnp.float32),
            mesh=mesh,
            compiler_params=pltpu.CompilerParams(
                needs_layout_passes=False, use_tc_tiling_on_sc=False),
            scratch_types=[
                pltpu.VMEM((N_CHUNKS, IDX_CHUNK), jnp.int32),
                pltpu.VMEM((N_CHUNKS, IDX_CHUNK), jnp.int32),
                pltpu.VMEM((B_PER_W, EMBED_DIM), jnp.float32),
                pltpu.VMEM((B_PER_W, EMBED_DIM), jnp.float32),
                pltpu.VMEM((B_PER_W,), jnp.float32),
                pltpu.SemaphoreType.DMA,
            ],
        )
        def k(centers_hbm, contexts_hbm, emb_hbm, out_hbm,
              cidx_v, xidx_v, u_v, v_v, score_v, sem):
            wid = lax.axis_index("s") * NUM_CORES + lax.axis_index("c")
            base = wid * B_PER_W

            pltpu.sync_copy(centers_hbm.at[pl.ds(wid * N_CHUNKS, N_CHUNKS)],
                            cidx_v)
            pltpu.sync_copy(contexts_hbm.at[pl.ds(wid * N_CHUNKS, N_CHUNKS)],
                            xidx_v)

            copies = []
            for j in range(N_CHUNKS):
                copies.append(pltpu.async_copy(
                    emb_hbm.at[cidx_v.at[j]],
                    u_v.at[pl.ds(j * IDX_CHUNK, IDX_CHUNK)], sem))
                copies.append(pltpu.async_copy(
                    emb_hbm.at[xidx_v.at[j]],
                    v_v.at[pl.ds(j * IDX_CHUNK, IDX_CHUNK)], sem))
            for c in copies:
                c.wait()

            lane = lax.iota(jnp.int32, LANES)

            def grp_body(g, _):
                rows = g * LANES + lane
                acc = jnp.zeros((LANES,), jnp.float32)
                for c in range(EMBED_DIM):
                    col = jnp.full((LANES,), c, jnp.int32)
                    un = plsc.load_gather(u_v, [rows, col])
                    vn = plsc.load_gather(v_v, [rows, col])
                    acc = acc + un * vn
                score_v[pl.ds(g * LANES, LANES)] = acc
                return ()

            lax.fori_loop(0, B_PER_W // LANES, grp_body, ())

            pltpu.sync_copy(score_v, out_hbm.at[pl.ds(base, B_PER_W)])

        return k(centers2d, contexts2d, emb)

    return run(centers2d, contexts2d, emb)


def _tc_loss(scores):
    x2d = scores.reshape(BATCH // 128, 128)

    def body(x_ref, o_ref):
        x = x_ref[...]
        ls = jnp.minimum(x, 0.0) - jnp.log1p(jnp.exp(-jnp.abs(x)))
        o_ref[0, 0] = -jnp.sum(ls)

    out = pl.pallas_call(
        body,
        out_shape=jax.ShapeDtypeStruct((1, 1), jnp.float32),
        out_specs=pl.BlockSpec(memory_space=pltpu.SMEM),
    )(x2d)
    return out.reshape(())


def kernel(centers, contexts, emb):
    centers2d = centers.astype(jnp.int32).reshape(BATCH // IDX_CHUNK, IDX_CHUNK)
    contexts2d = contexts.astype(jnp.int32).reshape(BATCH // IDX_CHUNK, IDX_CHUNK)
    scores = _sc_scores(centers2d, contexts2d, emb)
    return _tc_loss(scores)
